# Initial kernel scaffold; baseline (speedup 1.0000x reference)
import jax
import jax.numpy as jnp
from jax.experimental import pallas as pl


def kernel(x, embedding_weight):
    raise NotImplementedError("write your pallas kernel here")


# speedup vs baseline: 1.1146x; 1.1146x over previous
import functools
import math

import jax
import jax.numpy as jnp
from jax import lax
from jax.experimental import pallas as pl
from jax.experimental.pallas import tpu as pltpu
from jax.experimental.pallas import tpu_sc as plsc

D_MODEL = 768
SCALE = math.sqrt(D_MODEL)
NC, NS, LANES = 2, 16, 16
NW = NC * NS
CHUNK = 32


def _emb_body(nchunks, b_per_w, x_hbm, tab_hbm, out_hbm, idx_v, rows_v, sem):
    wid = lax.axis_index("s") * NC + lax.axis_index("c")
    base = wid * b_per_w
    pltpu.sync_copy(x_hbm.at[wid], idx_v)

    def chunk_body(j, carry):
        pltpu.async_copy(tab_hbm.at[idx_v.at[j]], rows_v, sem).wait()

        def row_body(r, c2):
            for c in range(D_MODEL // LANES):
                sl = pl.ds(c * LANES, LANES)
                rows_v[r, sl] = rows_v[r, sl] * SCALE
            return c2

        lax.fori_loop(0, CHUNK, row_body, 0)

        pltpu.sync_copy(rows_v, out_hbm.at[pl.ds(base + j * CHUNK, CHUNK)])
        return carry

    lax.fori_loop(0, nchunks, chunk_body, 0)


def kernel(x, embedding_weight):
    orig_shape = x.shape
    b_total = x.size
    b_per_w = b_total // NW
    nchunks = b_per_w // CHUNK
    x_resh = x.reshape(NW, nchunks, CHUNK).astype(jnp.int32)

    mesh = plsc.VectorSubcoreMesh(core_axis_name="c", subcore_axis_name="s")
    emb = pl.kernel(
        functools.partial(_emb_body, nchunks, b_per_w),
        out_type=jax.ShapeDtypeStruct((b_total, D_MODEL), jnp.float32),
        mesh=mesh,
        scratch_types=[
            pltpu.VMEM((nchunks, CHUNK), jnp.int32),
            pltpu.VMEM((CHUNK, D_MODEL), jnp.float32),
            pltpu.SemaphoreType.DMA,
        ],
    )
    out = emb(x_resh, embedding_weight)
    return out.reshape(orig_shape + (D_MODEL,))


# speedup vs baseline: 1.3908x; 1.2478x over previous
import functools
import math

import jax
import jax.numpy as jnp
from jax import lax
from jax.experimental import pallas as pl
from jax.experimental.pallas import tpu as pltpu
from jax.experimental.pallas import tpu_sc as plsc

D_MODEL = 768
SCALE = math.sqrt(D_MODEL)
NC, NS, LANES = 2, 16, 16
NW = NC * NS
CHUNK = 32
NBUF = 4


def _scale_buf(buf):
    def row_body(r, carry):
        for c in range(D_MODEL // LANES):
            sl = pl.ds(c * LANES, LANES)
            buf[r, sl] = buf[r, sl] * SCALE
        return carry

    lax.fori_loop(0, CHUNK, row_body, 0)


def _emb_body(nchunks, b_per_w, x_hbm, tab_hbm, out_hbm, idx_v, rows_v, *sems):
    gs, ss = sems[:NBUF], sems[NBUF:]
    wid = lax.axis_index("s") * NC + lax.axis_index("c")
    base = wid * b_per_w
    pltpu.sync_copy(x_hbm.at[wid], idx_v)

    def start_gather(j, b):
        pltpu.async_copy(tab_hbm.at[idx_v.at[j]], rows_v.at[b], gs[b])

    def wait_gather(b):
        pltpu.make_async_copy(tab_hbm.at[idx_v.at[0]], rows_v.at[b], gs[b]).wait()

    def start_store(j, b):
        dst = out_hbm.at[pl.ds(base + j * CHUNK, CHUNK)]
        pltpu.async_copy(rows_v.at[b], dst, ss[b])

    def wait_store(b):
        dst = out_hbm.at[pl.ds(base, CHUNK)]
        pltpu.make_async_copy(rows_v.at[b], dst, ss[b]).wait()

    for b in range(NBUF):
        start_gather(b, b)

    ngroups = nchunks // NBUF

    def group_body(g, carry):
        for b in range(NBUF):
            wait_gather(b)
            _scale_buf(rows_v.at[b])
            start_store(g * NBUF + b, b)
        for b in range(NBUF):
            wait_store(b)
            start_gather((g + 1) * NBUF + b, b)
        return carry

    lax.fori_loop(0, ngroups - 1, group_body, 0)

    g = ngroups - 1
    for b in range(NBUF):
        wait_gather(b)
        _scale_buf(rows_v.at[b])
        start_store(g * NBUF + b, b)
    for b in range(NBUF):
        wait_store(b)


def kernel(x, embedding_weight):
    orig_shape = x.shape
    b_total = x.size
    b_per_w = b_total // NW
    nchunks = b_per_w // CHUNK
    x_resh = x.reshape(NW, nchunks, CHUNK).astype(jnp.int32)

    mesh = plsc.VectorSubcoreMesh(core_axis_name="c", subcore_axis_name="s")
    emb = pl.kernel(
        functools.partial(_emb_body, nchunks, b_per_w),
        out_type=jax.ShapeDtypeStruct((b_total, D_MODEL), jnp.float32),
        mesh=mesh,
        scratch_types=[
            pltpu.VMEM((nchunks, CHUNK), jnp.int32),
            pltpu.VMEM((NBUF, CHUNK, D_MODEL), jnp.float32),
        ] + [pltpu.SemaphoreType.DMA] * (2 * NBUF),
    )
    out = emb(x_resh, embedding_weight)
    return out.reshape(orig_shape + (D_MODEL,))


# speedup vs baseline: 1.4410x; 1.0361x over previous
import functools
import math

import jax
import jax.numpy as jnp
from jax import lax
from jax.experimental import pallas as pl
from jax.experimental.pallas import tpu as pltpu
from jax.experimental.pallas import tpu_sc as plsc

D_MODEL = 768
SCALE = math.sqrt(D_MODEL)
NC, NS, LANES = 2, 16, 16
NW = NC * NS
CHUNK = 64
NBUF = 2


def _scale_buf(buf):
    def row_body(r, carry):
        for c in range(D_MODEL // LANES):
            sl = pl.ds(c * LANES, LANES)
            buf[r, sl] = buf[r, sl] * SCALE
        return carry

    lax.fori_loop(0, CHUNK, row_body, 0)


def _emb_body(nchunks, b_per_w, x_hbm, tab_hbm, out_hbm, idx_v, rows_v, *sems):
    gs, ss = sems[:NBUF], sems[NBUF:]
    wid = lax.axis_index("s") * NC + lax.axis_index("c")
    base = wid * b_per_w
    pltpu.sync_copy(x_hbm.at[wid], idx_v)

    def start_gather(j, b):
        pltpu.async_copy(tab_hbm.at[idx_v.at[j]], rows_v.at[b], gs[b])

    def wait_gather(b):
        pltpu.make_async_copy(tab_hbm.at[idx_v.at[0]], rows_v.at[b], gs[b]).wait()

    def start_store(j, b):
        dst = out_hbm.at[pl.ds(base + j * CHUNK, CHUNK)]
        pltpu.async_copy(rows_v.at[b], dst, ss[b])

    def wait_store(b):
        dst = out_hbm.at[pl.ds(base, CHUNK)]
        pltpu.make_async_copy(rows_v.at[b], dst, ss[b]).wait()

    for b in range(NBUF):
        start_gather(b, b)

    ngroups = nchunks // NBUF

    def group_body(g, carry):
        for b in range(NBUF):
            wait_gather(b)
            _scale_buf(rows_v.at[b])
            start_store(g * NBUF + b, b)
        for b in range(NBUF):
            wait_store(b)
            start_gather((g + 1) * NBUF + b, b)
        return carry

    lax.fori_loop(0, ngroups - 1, group_body, 0)

    g = ngroups - 1
    for b in range(NBUF):
        wait_gather(b)
        _scale_buf(rows_v.at[b])
        start_store(g * NBUF + b, b)
    for b in range(NBUF):
        wait_store(b)


def kernel(x, embedding_weight):
    orig_shape = x.shape
    b_total = x.size
    b_per_w = b_total // NW
    nchunks = b_per_w // CHUNK
    x_resh = x.reshape(NW, nchunks, CHUNK).astype(jnp.int32)

    mesh = plsc.VectorSubcoreMesh(core_axis_name="c", subcore_axis_name="s")
    emb = pl.kernel(
        functools.partial(_emb_body, nchunks, b_per_w),
        out_type=jax.ShapeDtypeStruct((b_total, D_MODEL), jnp.float32),
        mesh=mesh,
        scratch_types=[
            pltpu.VMEM((nchunks, CHUNK), jnp.int32),
            pltpu.VMEM((NBUF, CHUNK, D_MODEL), jnp.float32),
        ] + [pltpu.SemaphoreType.DMA] * (2 * NBUF),
    )
    out = emb(x_resh, embedding_weight)
    return out.reshape(orig_shape + (D_MODEL,))


# speedup vs baseline: 1.4790x; 1.0264x over previous
import functools
import math

import jax
import jax.numpy as jnp
from jax import lax
from jax.experimental import pallas as pl
from jax.experimental.pallas import tpu as pltpu
from jax.experimental.pallas import tpu_sc as plsc

D_MODEL = 768
SCALE = math.sqrt(D_MODEL)
NC, NS, LANES = 2, 16, 16
NW = NC * NS
CHUNK = 64
NBUF = 2


HALF = CHUNK // 2


def _scale_rows(buf, start, nrows):
    def row_body(r, carry):
        for c in range(D_MODEL // LANES):
            sl = pl.ds(c * LANES, LANES)
            buf[r, sl] = buf[r, sl] * SCALE
        return carry

    lax.fori_loop(start, start + nrows, row_body, 0)


def _emb_body(nchunks, b_per_w, x_hbm, tab_hbm, out_hbm, idx_v, rows_v, *sems):
    gs, ss = sems[:NBUF], sems[NBUF:]
    wid = lax.axis_index("s") * NC + lax.axis_index("c")
    base = wid * b_per_w
    pltpu.sync_copy(x_hbm.at[wid], idx_v)

    def start_gather(j, b):
        pltpu.async_copy(tab_hbm.at[idx_v.at[j]], rows_v.at[b], gs[b])

    def wait_gather(b):
        pltpu.make_async_copy(tab_hbm.at[idx_v.at[0]], rows_v.at[b], gs[b]).wait()

    def start_store_half(j, b, h):
        src = rows_v.at[b].at[pl.ds(h * HALF, HALF)]
        dst = out_hbm.at[pl.ds(base + j * CHUNK + h * HALF, HALF)]
        pltpu.async_copy(src, dst, ss[b])

    def wait_store(b):
        dst = out_hbm.at[pl.ds(base, CHUNK)]
        pltpu.make_async_copy(rows_v.at[b], dst, ss[b]).wait()

    for b in range(NBUF):
        start_gather(b, b)

    ngroups = nchunks // NBUF

    def group_body(g, carry):
        for b in range(NBUF):
            wait_gather(b)
            for h in range(2):
                _scale_rows(rows_v.at[b], h * HALF, HALF)
                start_store_half(g * NBUF + b, b, h)
        for b in range(NBUF):
            wait_store(b)
            start_gather((g + 1) * NBUF + b, b)
        return carry

    lax.fori_loop(0, ngroups - 1, group_body, 0)

    g = ngroups - 1
    for b in range(NBUF):
        wait_gather(b)
        for h in range(2):
            _scale_rows(rows_v.at[b], h * HALF, HALF)
            start_store_half(g * NBUF + b, b, h)
    for b in range(NBUF):
        wait_store(b)


def kernel(x, embedding_weight):
    orig_shape = x.shape
    b_total = x.size
    b_per_w = b_total // NW
    nchunks = b_per_w // CHUNK
    x_resh = x.reshape(NW, nchunks, CHUNK).astype(jnp.int32)

    mesh = plsc.VectorSubcoreMesh(core_axis_name="c", subcore_axis_name="s")
    emb = pl.kernel(
        functools.partial(_emb_body, nchunks, b_per_w),
        out_type=jax.ShapeDtypeStruct((b_total, D_MODEL), jnp.float32),
        mesh=mesh,
        scratch_types=[
            pltpu.VMEM((nchunks, CHUNK), jnp.int32),
            pltpu.VMEM((NBUF, CHUNK, D_MODEL), jnp.float32),
        ] + [pltpu.SemaphoreType.DMA] * (2 * NBUF),
    )
    out = emb(x_resh, embedding_weight)
    return out.reshape(orig_shape + (D_MODEL,))
